# initial kernel scaffold (unmeasured)
import jax
import jax.numpy as jnp
from jax import lax
from jax.experimental import pallas as pl
from jax.experimental.pallas import tpu as pltpu

N_DEV = 4
EPS = 1e-5


def kernel(x, gamma, beta):
    m, n_shard = x.shape
    n_global = n_shard * N_DEV
    pr, pc = m // 128, 128

    gamma2 = gamma.reshape(1, n_shard)
    beta2 = beta.reshape(1, n_shard)

    def body(x_ref, g_ref, b_ref, out_ref, comm_ref, send_sems, recv_sems):
        my = lax.axis_index("i")

        barrier_sem = pltpu.get_barrier_semaphore()
        for k in range(1, N_DEV):
            pl.semaphore_signal(
                barrier_sem, inc=1,
                device_id=(lax.rem(my + k, N_DEV),),
                device_id_type=pl.DeviceIdType.MESH,
            )
        pl.semaphore_wait(barrier_sem, N_DEV - 1)

        xf = x_ref[:, :]
        s1 = jnp.sum(xf, axis=1, keepdims=True)
        s2 = jnp.sum(xf * xf, axis=1, keepdims=True)
        comm_ref[0, 0] = s1.reshape(pr, pc)
        comm_ref[0, 1] = s2.reshape(pr, pc)

        rdmas = []
        for k in range(1, N_DEV):
            rdma = pltpu.make_async_remote_copy(
                src_ref=comm_ref.at[0],
                dst_ref=comm_ref.at[k],
                send_sem=send_sems.at[k - 1],
                recv_sem=recv_sems.at[k - 1],
                device_id=(lax.rem(my + k, N_DEV),),
                device_id_type=pl.DeviceIdType.MESH,
            )
            rdma.start()
            rdmas.append(rdma)

        for rdma in rdmas:
            rdma.wait_recv()
        for rdma in rdmas:
            rdma.wait_send()

        tot1 = comm_ref[0, 0] + comm_ref[1, 0] + comm_ref[2, 0] + comm_ref[3, 0]
        tot2 = comm_ref[0, 1] + comm_ref[1, 1] + comm_ref[2, 1] + comm_ref[3, 1]
        inv_n = 1.0 / n_global
        mean = (tot1 * inv_n).reshape(m, 1)
        var = (tot2 * inv_n).reshape(m, 1) - mean * mean
        rstd = lax.rsqrt(var + EPS)
        out_ref[:, :] = (
            (xf - mean) * rstd * g_ref[:, :] + b_ref[:, :]
        ).astype(out_ref.dtype)

    return pl.pallas_call(
        body,
        out_shape=jax.ShapeDtypeStruct((m, n_shard), jnp.bfloat16),
        in_specs=[
            pl.BlockSpec(memory_space=pltpu.VMEM),
            pl.BlockSpec(memory_space=pltpu.VMEM),
            pl.BlockSpec(memory_space=pltpu.VMEM),
        ],
        out_specs=pl.BlockSpec(memory_space=pltpu.VMEM),
        scratch_shapes=[
            pltpu.VMEM((N_DEV, 2, pr, pc), jnp.float32),
            pltpu.SemaphoreType.DMA((N_DEV - 1,)),
            pltpu.SemaphoreType.DMA((N_DEV - 1,)),
        ],
        compiler_params=pltpu.CompilerParams(collective_id=0),
    )(x, gamma2, beta2)


# baseline (device time: 15649 ns/iter reference)
import jax
import jax.numpy as jnp
from jax import lax
from jax.experimental import pallas as pl
from jax.experimental.pallas import tpu as pltpu

N_DEV = 4
EPS = 1e-5


def kernel(x, gamma, beta):
    m, n_shard = x.shape
    n_global = n_shard * N_DEV
    pr, pc = m // 128, 128

    gamma2 = gamma.reshape(1, n_shard)
    beta2 = beta.reshape(1, n_shard)

    def body(x_ref, g_ref, b_ref, out_ref, comm_ref, send_sems, recv_sems):
        my = lax.axis_index("i")

        barrier_sem = pltpu.get_barrier_semaphore()
        for k in range(1, N_DEV):
            pl.semaphore_signal(
                barrier_sem, inc=1,
                device_id=(lax.rem(my + k, N_DEV),),
                device_id_type=pl.DeviceIdType.MESH,
            )
        pl.semaphore_wait(barrier_sem, N_DEV - 1)

        row = lax.broadcasted_iota(jnp.int32, (m, pc), 0)
        lane = lax.broadcasted_iota(jnp.int32, (m, pc), 1)
        mask = (lane == row % pc).astype(jnp.float32)
        sel = (
            lax.broadcasted_iota(jnp.int32, (m, pr), 1)
            == lax.broadcasted_iota(jnp.int32, (m, pr), 0) // pc
        ).astype(jnp.float32)

        def pack(s):
            return lax.dot_general(
                sel, s * mask, (((0,), (0,)), ((), ())),
                preferred_element_type=jnp.float32,
            )

        def unpack(t):
            u = lax.dot_general(
                sel, t, (((1,), (0,)), ((), ())),
                preferred_element_type=jnp.float32,
            )
            return jnp.sum(u * mask, axis=1, keepdims=True)

        xf = x_ref[:, :]
        s1 = jnp.sum(xf, axis=1, keepdims=True)
        s2 = jnp.sum(xf * xf, axis=1, keepdims=True)
        comm_ref[0, 0] = pack(s1)
        comm_ref[0, 1] = pack(s2)

        rdmas = []
        for k in range(1, N_DEV):
            rdma = pltpu.make_async_remote_copy(
                src_ref=comm_ref.at[0],
                dst_ref=comm_ref.at[k],
                send_sem=send_sems.at[k - 1],
                recv_sem=recv_sems.at[k - 1],
                device_id=(lax.rem(my + k, N_DEV),),
                device_id_type=pl.DeviceIdType.MESH,
            )
            rdma.start()
            rdmas.append(rdma)

        for rdma in rdmas:
            rdma.wait_recv()
        for rdma in rdmas:
            rdma.wait_send()

        tot1 = comm_ref[0, 0] + comm_ref[1, 0] + comm_ref[2, 0] + comm_ref[3, 0]
        tot2 = comm_ref[0, 1] + comm_ref[1, 1] + comm_ref[2, 1] + comm_ref[3, 1]
        inv_n = 1.0 / n_global
        mean = unpack(tot1) * inv_n
        var = unpack(tot2) * inv_n - mean * mean
        rstd = lax.rsqrt(var + EPS)
        out_ref[:, :] = (
            (xf - mean) * rstd * g_ref[:, :] + b_ref[:, :]
        ).astype(out_ref.dtype)

    return pl.pallas_call(
        body,
        out_shape=jax.ShapeDtypeStruct((m, n_shard), jnp.bfloat16),
        in_specs=[
            pl.BlockSpec(memory_space=pltpu.VMEM),
            pl.BlockSpec(memory_space=pltpu.VMEM),
            pl.BlockSpec(memory_space=pltpu.VMEM),
        ],
        out_specs=pl.BlockSpec(memory_space=pltpu.VMEM),
        scratch_shapes=[
            pltpu.VMEM((N_DEV, 2, pr, pc), jnp.float32),
            pltpu.SemaphoreType.DMA((N_DEV - 1,)),
            pltpu.SemaphoreType.DMA((N_DEV - 1,)),
        ],
        compiler_params=pltpu.CompilerParams(collective_id=0),
    )(x, gamma2, beta2)


# device time: 10533 ns/iter; 1.4857x vs baseline; 1.4857x over previous
import os

import jax
import jax.numpy as jnp
from jax import lax
from jax.experimental import pallas as pl
from jax.experimental.pallas import tpu as pltpu

N_DEV = 4
EPS = 1e-5
_NO_COMM = os.environ.get("KERNEL_NO_COMM", "0") == "1"


def kernel(x, gamma, beta):
    m, n_shard = x.shape
    n_global = n_shard * N_DEV
    pr, pc = m // 128, 128

    gamma2 = gamma.reshape(1, n_shard)
    beta2 = beta.reshape(1, n_shard)

    def body(x_ref, g_ref, b_ref, out_ref, comm_ref, send_sems, recv_sems):
        my = lax.axis_index("i")

        if not _NO_COMM:
            barrier_sem = pltpu.get_barrier_semaphore()
            for k in range(1, N_DEV):
                pl.semaphore_signal(
                    barrier_sem, inc=1,
                    device_id=(lax.rem(my + k, N_DEV),),
                    device_id_type=pl.DeviceIdType.MESH,
                )
            pl.semaphore_wait(barrier_sem, N_DEV - 1)

        row = lax.broadcasted_iota(jnp.int32, (m, pc), 0)
        lane = lax.broadcasted_iota(jnp.int32, (m, pc), 1)
        mask = (lane == row % pc).astype(jnp.float32)
        sel = (
            lax.broadcasted_iota(jnp.int32, (m, pr), 1)
            == lax.broadcasted_iota(jnp.int32, (m, pr), 0) // pc
        ).astype(jnp.float32)

        def pack(s):
            return lax.dot_general(
                sel, s * mask, (((0,), (0,)), ((), ())),
                preferred_element_type=jnp.float32,
            )

        def unpack(t):
            u = lax.dot_general(
                sel, t, (((1,), (0,)), ((), ())),
                preferred_element_type=jnp.float32,
            )
            return jnp.sum(u * mask, axis=1, keepdims=True)

        xf = x_ref[:, :]
        s1 = jnp.sum(xf, axis=1, keepdims=True)
        s2 = jnp.sum(xf * xf, axis=1, keepdims=True)
        comm_ref[0, 0] = pack(s1)
        comm_ref[0, 1] = pack(s2)

        if not _NO_COMM:
            rdmas = []
            for k in range(1, N_DEV):
                rdma = pltpu.make_async_remote_copy(
                    src_ref=comm_ref.at[0],
                    dst_ref=comm_ref.at[k],
                    send_sem=send_sems.at[k - 1],
                    recv_sem=recv_sems.at[k - 1],
                    device_id=(lax.rem(my + k, N_DEV),),
                    device_id_type=pl.DeviceIdType.MESH,
                )
                rdma.start()
                rdmas.append(rdma)

            for rdma in rdmas:
                rdma.wait_recv()
            for rdma in rdmas:
                rdma.wait_send()

            tot1 = (comm_ref[0, 0] + comm_ref[1, 0]
                    + comm_ref[2, 0] + comm_ref[3, 0])
            tot2 = (comm_ref[0, 1] + comm_ref[1, 1]
                    + comm_ref[2, 1] + comm_ref[3, 1])
        else:
            tot1 = comm_ref[0, 0] * 4.0
            tot2 = comm_ref[0, 1] * 4.0
        inv_n = 1.0 / n_global
        mean = unpack(tot1) * inv_n
        var = unpack(tot2) * inv_n - mean * mean
        rstd = lax.rsqrt(var + EPS)
        out_ref[:, :] = (
            (xf - mean) * rstd * g_ref[:, :] + b_ref[:, :]
        ).astype(out_ref.dtype)

    return pl.pallas_call(
        body,
        out_shape=jax.ShapeDtypeStruct((m, n_shard), jnp.bfloat16),
        in_specs=[
            pl.BlockSpec(memory_space=pltpu.VMEM),
            pl.BlockSpec(memory_space=pltpu.VMEM),
            pl.BlockSpec(memory_space=pltpu.VMEM),
        ],
        out_specs=pl.BlockSpec(memory_space=pltpu.VMEM),
        scratch_shapes=[
            pltpu.VMEM((N_DEV, 2, pr, pc), jnp.float32),
            pltpu.SemaphoreType.DMA((N_DEV - 1,)),
            pltpu.SemaphoreType.DMA((N_DEV - 1,)),
        ],
        compiler_params=(
            pltpu.CompilerParams()
            if _NO_COMM
            else pltpu.CompilerParams(collective_id=0)
        ),
    )(x, gamma2, beta2)
